# baseline (speedup 1.0000x reference)
import functools

import jax
import jax.numpy as jnp
from jax import lax
from jax.experimental import pallas as pl
from jax.experimental.pallas import tpu as pltpu
from jax.experimental.pallas import tpu_sc as plsc

NC = 2
NS = 16
NW = NC * NS
CHUNK = 128


def _prop_body(xws, srcp, dstp, zrows, out, srcv, dstv, rows, isems, gsems,
               acc_sh):
    c = lax.axis_index("c")
    s = lax.axis_index("s")
    npad = acc_sh.shape[0]
    rows_per_tile = npad // NS
    ew = srcp.shape[0] // NW
    nch = ew // CHUNK
    base = (c * NS + s) * ew

    pltpu.sync_copy(zrows, acc_sh.at[pl.ds(s * rows_per_tile, rows_per_tile)])
    plsc.subcore_barrier()

    def idx_issue(j, b):
        off = base + j * CHUNK
        pltpu.async_copy(srcp.at[pl.ds(off, CHUNK)], srcv[b], isems[b])
        pltpu.async_copy(dstp.at[pl.ds(off, CHUNK)], dstv[b], isems[b])

    def idx_wait(j, b):
        off = base + j * CHUNK
        pltpu.make_async_copy(srcp.at[pl.ds(off, CHUNK)], srcv[b], isems[b]).wait()
        pltpu.make_async_copy(dstp.at[pl.ds(off, CHUNK)], dstv[b], isems[b]).wait()

    def gather_issue(j, b):
        pltpu.async_copy(xws.at[srcv[b]], rows[b], gsems[b])

    def gather_wait(j, b):
        pltpu.make_async_copy(xws.at[srcv[b]], rows[b], gsems[b]).wait()

    def scatter(j, b):
        pltpu.sync_copy(rows[b], acc_sh.at[dstv[b]], add=True)

    def step(j, b, issue_next_gather, issue_next2_idx):
        if issue_next_gather:
            idx_wait(j + 1, 1 - b)
            gather_issue(j + 1, 1 - b)
        gather_wait(j, b)
        scatter(j, b)
        if issue_next2_idx:
            idx_issue(j + 2, b)

    idx_issue(0, 0)
    idx_issue(1, 1)
    idx_wait(0, 0)
    gather_issue(0, 0)

    def group(g, _):
        j0 = g * 2
        step(j0, 0, True, True)
        step(j0 + 1, 1, True, True)
        return 0

    lax.fori_loop(0, nch // 2 - 1, group, 0)
    step(nch - 2, 0, True, False)
    step(nch - 1, 1, False, False)

    plsc.subcore_barrier()
    pltpu.sync_copy(acc_sh.at[pl.ds(s * rows_per_tile, rows_per_tile)],
                    out.at[c, pl.ds(s * rows_per_tile, rows_per_tile)])


def _deg_body(dstp, ones, zvec, out, dst_v, ones_v, acc_sh):
    c = lax.axis_index("c")
    s = lax.axis_index("s")
    npad = acc_sh.shape[0]
    per_tile = npad // NS
    ew = dstp.shape[0] // NW
    nch = ew // CHUNK
    base = (c * NS + s) * ew

    pltpu.sync_copy(zvec, acc_sh.at[pl.ds(s * per_tile, per_tile)])
    pltpu.sync_copy(ones, ones_v)
    plsc.subcore_barrier()

    def chunk(k, _):
        off = base + k * CHUNK
        pltpu.sync_copy(dstp.at[pl.ds(off, CHUNK)], dst_v)
        pltpu.sync_copy(ones_v, acc_sh.at[dst_v], add=True)
        return 0

    lax.fori_loop(0, nch, chunk, 0)
    plsc.subcore_barrier()
    pltpu.sync_copy(acc_sh.at[pl.ds(s * per_tile, per_tile)],
                    out.at[c, pl.ds(s * per_tile, per_tile)])


def _mm1_body(x_ref, w_ref, d0_ref, d1_ref, out_ref):
    dis = lax.rsqrt(d0_ref[...] + d1_ref[...] + 1.0)
    xw = jnp.dot(x_ref[...], w_ref[...], preferred_element_type=jnp.float32)
    out_ref[...] = dis * xw


def _mm2_body(p0_ref, p1_ref, xws_ref, b_ref, w_ref, d0_ref, d1_ref, out_ref):
    dis = lax.rsqrt(d0_ref[...] + d1_ref[...] + 1.0)
    xws = xws_ref[...]
    h = dis * (p0_ref[...] + p1_ref[...] + xws) + b_ref[...]
    h = jnp.maximum(h, 0.0)
    hw = jnp.dot(h, w_ref[...], preferred_element_type=jnp.float32)
    out_ref[...] = dis * hw


def _mm3_body(p0_ref, p1_ref, xws_ref, b_ref, w_ref, bc_ref, d0_ref, d1_ref,
              out_ref):
    dis = lax.rsqrt(d0_ref[...] + d1_ref[...] + 1.0)
    h2 = dis * (p0_ref[...] + p1_ref[...] + xws_ref[...]) + b_ref[...]
    out_ref[...] = jnp.dot(h2, w_ref[...],
                           preferred_element_type=jnp.float32) + bc_ref[...]


def kernel(x, edge_index, W1, b1, W2, b2, Wc, bc):
    N, D = x.shape
    H = W1.shape[1]
    C = Wc.shape[1]
    E = edge_index.shape[1]
    npad = ((N + 16 * CHUNK - 1) // (16 * CHUNK)) * (16 * CHUNK)
    eq = NW * CHUNK * 8
    epad = ((E + eq - 1) // eq) * eq

    src = edge_index[0]
    dst = edge_index[1]
    srcp = jnp.concatenate([src, jnp.zeros((epad - E,), jnp.int32)])
    junk = N + (jnp.arange(epad - E, dtype=jnp.int32) % (npad - N))
    dstp = jnp.concatenate([dst, junk])
    srcp2 = srcp.reshape(epad // CHUNK, CHUNK)
    dstp2 = dstp.reshape(epad // CHUNK, CHUNK)
    nch = epad // CHUNK // NW

    rows_per_tile = npad // NS
    zrows = jnp.zeros((rows_per_tile, D), jnp.float32)
    zvec = jnp.zeros((rows_per_tile,), jnp.float32)
    ones = jnp.ones((CHUNK,), jnp.float32)

    mesh = plsc.VectorSubcoreMesh(core_axis_name="c", subcore_axis_name="s")

    deg_k = functools.partial(
        pl.kernel,
        out_type=jax.ShapeDtypeStruct((NC, npad), jnp.float32),
        mesh=mesh,
        scratch_types=[
            pltpu.VMEM((CHUNK,), jnp.int32),
            pltpu.VMEM((CHUNK,), jnp.float32),
            pltpu.VMEM_SHARED((npad,), jnp.float32),
        ],
    )(_deg_body)
    degp = deg_k(dstp, ones, zvec)

    prop_k = functools.partial(
        pl.kernel,
        out_type=jax.ShapeDtypeStruct((NC, npad, D), jnp.float32),
        mesh=mesh,
        scratch_types=[
            [pltpu.VMEM((CHUNK,), jnp.int32) for _ in range(2)],
            [pltpu.VMEM((CHUNK,), jnp.int32) for _ in range(2)],
            [pltpu.VMEM((CHUNK, D), jnp.float32) for _ in range(2)],
            [pltpu.SemaphoreType.DMA for _ in range(2)],
            [pltpu.SemaphoreType.DMA for _ in range(2)],
            pltpu.VMEM_SHARED((npad, D), jnp.float32),
        ],
    )(_prop_body)

    d0 = degp[0, :N].reshape(N, 1)
    d1 = degp[1, :N].reshape(N, 1)

    xws1 = pl.pallas_call(
        _mm1_body,
        out_shape=jax.ShapeDtypeStruct((N, H), jnp.float32),
    )(x, W1, d0, d1)

    P1 = prop_k(xws1, srcp, dstp, zrows)

    xws2 = pl.pallas_call(
        _mm2_body,
        out_shape=jax.ShapeDtypeStruct((N, H), jnp.float32),
    )(P1[0, :N], P1[1, :N], xws1, b1.reshape(1, H), W2, d0, d1)

    P2 = prop_k(xws2, srcp, dstp, zrows)

    cpad = 128
    Wc_p = jnp.pad(Wc, ((0, 0), (0, cpad - C)))
    bc_p = jnp.pad(bc, (0, cpad - C))
    out_p = pl.pallas_call(
        _mm3_body,
        out_shape=jax.ShapeDtypeStruct((N, cpad), jnp.float32),
    )(P2[0, :N], P2[1, :N], xws2, b2.reshape(1, H), Wc_p,
      bc_p.reshape(1, cpad), d0, d1)

    return out_p[:, :C]


# speedup vs baseline: 1.2151x; 1.2151x over previous
import functools

import jax
import jax.numpy as jnp
from jax import lax
from jax.experimental import pallas as pl
from jax.experimental.pallas import tpu as pltpu
from jax.experimental.pallas import tpu_sc as plsc

NC = 2
NS = 16
NW = NC * NS
CHUNK = 128


def _prop_body(xws, epk3, zrows, out, ebuf, rows_v, sem, acc_sh):
    c = lax.axis_index("c")
    s = lax.axis_index("s")
    npad = acc_sh.shape[0]
    rows_per_tile = npad // NS
    nch = epk3.shape[0] // NW
    base = (c * NS + s) * nch

    pltpu.sync_copy(zrows, acc_sh.at[pl.ds(s * rows_per_tile, rows_per_tile)])
    plsc.subcore_barrier()

    def chunk(k, _):
        pltpu.sync_copy(epk3.at[base + k], ebuf)
        pltpu.async_copy(xws.at[ebuf.at[0]], rows_v, sem).wait()
        pltpu.sync_copy(rows_v, acc_sh.at[ebuf.at[1]], add=True)
        return 0

    lax.fori_loop(0, nch, chunk, 0)
    plsc.subcore_barrier()
    pltpu.sync_copy(acc_sh.at[pl.ds(s * rows_per_tile, rows_per_tile)],
                    out.at[c, pl.ds(s * rows_per_tile, rows_per_tile)])


def _deg_body(dstp, ones, zvec, out, dst_v, ones_v, acc_sh):
    c = lax.axis_index("c")
    s = lax.axis_index("s")
    npad = acc_sh.shape[0]
    per_tile = npad // NS
    ew = dstp.shape[0] // NW
    nch = ew // CHUNK
    base = (c * NS + s) * ew

    pltpu.sync_copy(zvec, acc_sh.at[pl.ds(s * per_tile, per_tile)])
    pltpu.sync_copy(ones, ones_v)
    plsc.subcore_barrier()

    def chunk(k, _):
        off = base + k * CHUNK
        pltpu.sync_copy(dstp.at[pl.ds(off, CHUNK)], dst_v)
        pltpu.sync_copy(ones_v, acc_sh.at[dst_v], add=True)
        return 0

    lax.fori_loop(0, nch, chunk, 0)
    plsc.subcore_barrier()
    pltpu.sync_copy(acc_sh.at[pl.ds(s * per_tile, per_tile)],
                    out.at[c, pl.ds(s * per_tile, per_tile)])


def _mm1_body(x_ref, w_ref, d0_ref, d1_ref, out_ref):
    dis = lax.rsqrt(d0_ref[...] + d1_ref[...] + 1.0)
    xw = jnp.dot(x_ref[...], w_ref[...], preferred_element_type=jnp.float32)
    out_ref[...] = dis * xw


def _mm2_body(p0_ref, p1_ref, xws_ref, b_ref, w_ref, d0_ref, d1_ref, out_ref):
    dis = lax.rsqrt(d0_ref[...] + d1_ref[...] + 1.0)
    xws = xws_ref[...]
    h = dis * (p0_ref[...] + p1_ref[...] + xws) + b_ref[...]
    h = jnp.maximum(h, 0.0)
    hw = jnp.dot(h, w_ref[...], preferred_element_type=jnp.float32)
    out_ref[...] = dis * hw


def _mm3_body(p0_ref, p1_ref, xws_ref, b_ref, w_ref, bc_ref, d0_ref, d1_ref,
              out_ref):
    dis = lax.rsqrt(d0_ref[...] + d1_ref[...] + 1.0)
    h2 = dis * (p0_ref[...] + p1_ref[...] + xws_ref[...]) + b_ref[...]
    out_ref[...] = jnp.dot(h2, w_ref[...],
                           preferred_element_type=jnp.float32) + bc_ref[...]


def kernel(x, edge_index, W1, b1, W2, b2, Wc, bc):
    N, D = x.shape
    H = W1.shape[1]
    C = Wc.shape[1]
    E = edge_index.shape[1]
    npad = ((N + 16 * CHUNK - 1) // (16 * CHUNK)) * (16 * CHUNK)
    epad = ((E + NW * CHUNK - 1) // (NW * CHUNK)) * (NW * CHUNK)

    src = edge_index[0]
    dst = edge_index[1]
    srcp = jnp.concatenate([src, jnp.zeros((epad - E,), jnp.int32)])
    junk = N + (jnp.arange(epad - E, dtype=jnp.int32) % (npad - N))
    dstp = jnp.concatenate([dst, junk])
    srcp2 = srcp.reshape(epad // CHUNK, CHUNK)
    dstp2 = dstp.reshape(epad // CHUNK, CHUNK)
    epk3 = jnp.stack([srcp2, dstp2], axis=1)
    nch = epad // CHUNK // NW

    rows_per_tile = npad // NS
    zrows = jnp.zeros((rows_per_tile, D), jnp.float32)
    zvec = jnp.zeros((rows_per_tile,), jnp.float32)
    ones = jnp.ones((CHUNK,), jnp.float32)

    mesh = plsc.VectorSubcoreMesh(core_axis_name="c", subcore_axis_name="s")

    deg_k = functools.partial(
        pl.kernel,
        out_type=jax.ShapeDtypeStruct((NC, npad), jnp.float32),
        mesh=mesh,
        scratch_types=[
            pltpu.VMEM((CHUNK,), jnp.int32),
            pltpu.VMEM((CHUNK,), jnp.float32),
            pltpu.VMEM_SHARED((npad,), jnp.float32),
        ],
    )(_deg_body)
    degp = deg_k(dstp, ones, zvec)

    prop_k = functools.partial(
        pl.kernel,
        out_type=jax.ShapeDtypeStruct((NC, npad, D), jnp.float32),
        mesh=mesh,
        scratch_types=[
            pltpu.VMEM((2, CHUNK), jnp.int32),
            pltpu.VMEM((CHUNK, D), jnp.float32),
            pltpu.SemaphoreType.DMA,
            pltpu.VMEM_SHARED((npad, D), jnp.float32),
        ],
    )(_prop_body)

    d0 = degp[0, :N].reshape(N, 1)
    d1 = degp[1, :N].reshape(N, 1)

    xws1 = pl.pallas_call(
        _mm1_body,
        out_shape=jax.ShapeDtypeStruct((N, H), jnp.float32),
    )(x, W1, d0, d1)

    P1 = prop_k(xws1, epk3, zrows)

    xws2 = pl.pallas_call(
        _mm2_body,
        out_shape=jax.ShapeDtypeStruct((N, H), jnp.float32),
    )(P1[0, :N], P1[1, :N], xws1, b1.reshape(1, H), W2, d0, d1)

    P2 = prop_k(xws2, epk3, zrows)

    cpad = 128
    Wc_p = jnp.pad(Wc, ((0, 0), (0, cpad - C)))
    bc_p = jnp.pad(bc, (0, cpad - C))
    out_p = pl.pallas_call(
        _mm3_body,
        out_shape=jax.ShapeDtypeStruct((N, cpad), jnp.float32),
    )(P2[0, :N], P2[1, :N], xws2, b2.reshape(1, H), Wc_p,
      bc_p.reshape(1, cpad), d0, d1)

    return out_p[:, :C]
